# baseline (speedup 1.0000x reference)
import jax
import jax.numpy as jnp
from jax.experimental import pallas as pl
from jax.experimental.pallas import tpu as pltpu

DIM_CODES = 64
DICT_SIZE = 8192
DIM_EMBED = 32
BATCH = 128
C_GRP = 8
N_GRP = DIM_CODES // C_GRP
K_Q = DICT_SIZE // 4
N_BUF = 5


def _fused_body(xt_ref, d_ref, idx_ref, ce_ref, oh_hbm,
                b0, b1, b2, b3, b4, sem):
    bufs = (b0, b1, b2, b3, b4)
    c = pl.program_id(0)
    g = c // C_GRP
    s = jax.lax.rem(c, C_GRP)
    gm = jax.lax.rem(g, N_BUF)

    def _quarter_dst(q):
        return oh_hbm.at[:, pl.ds(g * C_GRP, C_GRP), pl.ds(q * K_Q, K_Q)]

    for p in range(N_BUF):
        @pl.when((s == 0) & (gm == p))
        def _claim(p=p):
            for q in range(4):
                buf = (4 * p + q) % N_BUF

                @pl.when(4 * g + q >= N_BUF)
                def _wait_one(buf=buf, q=q):
                    pltpu.make_async_copy(bufs[buf], _quarter_dst(q),
                                          sem.at[buf]).wait()

    xt = xt_ref[0]
    dc = d_ref[0]
    xyT = jax.lax.dot_general(dc, xt, (((1,), (0,)), ((), ())),
                              preferred_element_type=jnp.float32)
    y_sq = jnp.sum(dc * dc, axis=1, keepdims=True)
    x_sq = jnp.sum(xt * xt, axis=0, keepdims=True)
    distT = x_sq - 2.0 * xyT + y_sq
    m = jnp.min(distT, axis=0, keepdims=True)
    kio = jax.lax.broadcasted_iota(jnp.int32, (DICT_SIZE, BATCH), 0)
    cand = jnp.where(distT == m, kio, DICT_SIZE)
    idxv = jnp.min(cand, axis=0, keepdims=True)
    idx_ref[0] = idxv
    onehotT = (kio == idxv).astype(jnp.float32)
    ceT = jax.lax.dot_general(dc, onehotT, (((0,), (0,)), ((), ())),
                              preferred_element_type=jnp.float32)
    ce_ref[0] = ceT

    idx_col = jnp.transpose(idxv)
    kio2 = jax.lax.broadcasted_iota(jnp.int32, (BATCH, DICT_SIZE), 1)
    oh = (kio2 == idx_col).astype(jnp.float32)
    for p in range(N_BUF):
        @pl.when(gm == p)
        def _store(p=p):
            for q in range(4):
                buf = (4 * p + q) % N_BUF
                bufs[buf][:, s, :] = oh[:, q * K_Q:(q + 1) * K_Q]

    for p in range(N_BUF):
        @pl.when((s == C_GRP - 1) & (gm == p))
        def _send(p=p):
            for q in range(4):
                buf = (4 * p + q) % N_BUF
                pltpu.make_async_copy(bufs[buf], _quarter_dst(q),
                                      sem.at[buf]).start()

    @pl.when(c == DIM_CODES - 1)
    def _drain():
        for qg in range(4 * N_GRP - N_BUF, 4 * N_GRP):
            gq, q = qg // 4, qg % 4
            buf = qg % N_BUF
            pltpu.make_async_copy(
                bufs[buf],
                oh_hbm.at[:, pl.ds(gq * C_GRP, C_GRP), pl.ds(q * K_Q, K_Q)],
                sem.at[buf]).wait()


def kernel(x, dictionary):
    xt = x.reshape(BATCH, DIM_CODES, DIM_EMBED).transpose(1, 2, 0)

    idx_t, ce_t, one_hot = pl.pallas_call(
        _fused_body,
        grid=(DIM_CODES,),
        in_specs=[
            pl.BlockSpec((1, DIM_EMBED, BATCH), lambda c: (c, 0, 0)),
            pl.BlockSpec((1, DICT_SIZE, DIM_EMBED), lambda c: (c, 0, 0)),
        ],
        out_specs=[
            pl.BlockSpec((1, 1, BATCH), lambda c: (c, 0, 0)),
            pl.BlockSpec((1, DIM_EMBED, BATCH), lambda c: (c, 0, 0)),
            pl.BlockSpec(memory_space=pltpu.MemorySpace.HBM),
        ],
        out_shape=[
            jax.ShapeDtypeStruct((DIM_CODES, 1, BATCH), jnp.int32),
            jax.ShapeDtypeStruct((DIM_CODES, DIM_EMBED, BATCH), jnp.float32),
            jax.ShapeDtypeStruct((BATCH, DIM_CODES, DICT_SIZE), jnp.float32),
        ],
        scratch_shapes=(
            [pltpu.VMEM((BATCH, C_GRP, K_Q), jnp.float32) for _ in range(N_BUF)]
            + [pltpu.SemaphoreType.DMA((N_BUF,))]
        ),
        compiler_params=pltpu.CompilerParams(
            vmem_limit_bytes=63 * 1024 * 1024,
        ),
    )(xt, dictionary)

    cw_e = ce_t.transpose(2, 0, 1).reshape(BATCH, DIM_CODES * DIM_EMBED)
    return cw_e, cw_e, one_hot


# speedup vs baseline: 1.2564x; 1.2564x over previous
import jax
import jax.numpy as jnp
from jax.experimental import pallas as pl
from jax.experimental.pallas import tpu as pltpu

DIM_CODES = 64
DICT_SIZE = 8192
DIM_EMBED = 32
BATCH = 128
C_GRP = 8
N_GRP = DIM_CODES // C_GRP
K_Q = DICT_SIZE // 4
N_BUF = 5


def _fused_body(xt_ref, d_ref, idx_ref, ce_ref, oh_hbm,
                b0, b1, b2, b3, b4, idxc_ref, sem):
    bufs = (b0, b1, b2, b3, b4)
    c = pl.program_id(0)
    g = c // C_GRP
    s = jax.lax.rem(c, C_GRP)
    gm = jax.lax.rem(g, N_BUF)

    def _quarter_dst(q):
        return oh_hbm.at[:, pl.ds(g * C_GRP, C_GRP), pl.ds(q * K_Q, K_Q)]

    xt = xt_ref[0]
    dc = d_ref[0]
    xyT = jax.lax.dot_general(dc, xt, (((1,), (0,)), ((), ())),
                              preferred_element_type=jnp.float32)
    y_sq = jnp.sum(dc * dc, axis=1, keepdims=True)
    x_sq = jnp.sum(xt * xt, axis=0, keepdims=True)
    distT = x_sq - 2.0 * xyT + y_sq
    m = jnp.min(distT, axis=0, keepdims=True)
    kio = jax.lax.broadcasted_iota(jnp.int32, (DICT_SIZE, BATCH), 0)
    cand = jnp.where(distT == m, kio, DICT_SIZE)
    idxv = jnp.min(cand, axis=0, keepdims=True)
    idx_ref[0] = idxv
    idxc_ref[pl.ds(s, 1), :] = idxv
    onehotT = (kio == idxv).astype(jnp.float32)
    ceT = jax.lax.dot_general(dc, onehotT, (((0,), (0,)), ((), ())),
                              preferred_element_type=jnp.float32)
    ce_ref[0] = ceT

    for p in range(N_BUF):
        @pl.when((s == C_GRP - 1) & (gm == p))
        def _flush(p=p):
            idxg = jnp.transpose(idxc_ref[...])
            for q in range(4):
                buf = (4 * p + q) % N_BUF

                @pl.when(4 * g + q >= N_BUF)
                def _wait_one(buf=buf, q=q):
                    pltpu.make_async_copy(bufs[buf], _quarter_dst(q),
                                          sem.at[buf]).wait()

                kio3 = jax.lax.broadcasted_iota(
                    jnp.int32, (BATCH, C_GRP, K_Q), 2) + (q * K_Q)
                bufs[buf][...] = (kio3 == idxg[:, :, None]).astype(jnp.float32)
                pltpu.make_async_copy(bufs[buf], _quarter_dst(q),
                                      sem.at[buf]).start()

    @pl.when(c == DIM_CODES - 1)
    def _drain():
        for qg in range(4 * N_GRP - N_BUF, 4 * N_GRP):
            gq, q = qg // 4, qg % 4
            buf = qg % N_BUF
            pltpu.make_async_copy(
                bufs[buf],
                oh_hbm.at[:, pl.ds(gq * C_GRP, C_GRP), pl.ds(q * K_Q, K_Q)],
                sem.at[buf]).wait()


def kernel(x, dictionary):
    xt = x.reshape(BATCH, DIM_CODES, DIM_EMBED).transpose(1, 2, 0)

    idx_t, ce_t, one_hot = pl.pallas_call(
        _fused_body,
        grid=(DIM_CODES,),
        in_specs=[
            pl.BlockSpec((1, DIM_EMBED, BATCH), lambda c: (c, 0, 0)),
            pl.BlockSpec((1, DICT_SIZE, DIM_EMBED), lambda c: (c, 0, 0)),
        ],
        out_specs=[
            pl.BlockSpec((1, 1, BATCH), lambda c: (c, 0, 0)),
            pl.BlockSpec((1, DIM_EMBED, BATCH), lambda c: (c, 0, 0)),
            pl.BlockSpec(memory_space=pltpu.MemorySpace.HBM),
        ],
        out_shape=[
            jax.ShapeDtypeStruct((DIM_CODES, 1, BATCH), jnp.int32),
            jax.ShapeDtypeStruct((DIM_CODES, DIM_EMBED, BATCH), jnp.float32),
            jax.ShapeDtypeStruct((BATCH, DIM_CODES, DICT_SIZE), jnp.float32),
        ],
        scratch_shapes=(
            [pltpu.VMEM((BATCH, C_GRP, K_Q), jnp.float32) for _ in range(N_BUF)]
            + [pltpu.VMEM((C_GRP, BATCH), jnp.int32),
               pltpu.SemaphoreType.DMA((N_BUF,))]
        ),
        compiler_params=pltpu.CompilerParams(
            vmem_limit_bytes=63 * 1024 * 1024,
        ),
    )(xt, dictionary)

    cw_e = ce_t.transpose(2, 0, 1).reshape(BATCH, DIM_CODES * DIM_EMBED)
    return cw_e, cw_e, one_hot


# speedup vs baseline: 1.3602x; 1.0826x over previous
import jax
import jax.numpy as jnp
from jax.experimental import pallas as pl
from jax.experimental.pallas import tpu as pltpu

DIM_CODES = 64
DICT_SIZE = 8192
DIM_EMBED = 32
BATCH = 128
N_SLOT = 4
K_Q = DICT_SIZE // 4


def _fused_body(xt_ref, d_ref, idx_ref, ce_ref, oh_hbm, oh_ref, kio_ref, sem):
    c = pl.program_id(0)
    slot = jax.lax.rem(c, N_SLOT)

    @pl.when(c == 0)
    def _init_iota():
        kio_ref[...] = jax.lax.broadcasted_iota(
            jnp.int32, (DICT_SIZE, BATCH), 0)

    def _copies(cc, sl):
        return [
            pltpu.make_async_copy(
                oh_ref.at[sl, :, pl.ds(q * K_Q, K_Q)],
                oh_hbm.at[:, cc, pl.ds(q * K_Q, K_Q)],
                sem.at[sl, q])
            for q in range(4)
        ]

    @pl.when(c >= N_SLOT)
    def _wait_prev():
        for cp in _copies(c - N_SLOT, slot):
            cp.wait()

    xt = xt_ref[0]
    dc = d_ref[0]
    xyT = jax.lax.dot_general(dc, xt, (((1,), (0,)), ((), ())),
                              preferred_element_type=jnp.float32)
    y_sq = jnp.sum(dc * dc, axis=1, keepdims=True)
    x_sq = jnp.sum(xt * xt, axis=0, keepdims=True)
    distT = x_sq - 2.0 * xyT + y_sq
    m = jnp.min(distT, axis=0, keepdims=True)
    kio = kio_ref[...]
    cand = jnp.where(distT == m, kio, DICT_SIZE)
    idxv = jnp.min(cand, axis=0, keepdims=True)
    idx_ref[0] = idxv
    onehotT = (kio == idxv).astype(jnp.float32)
    ceT = jax.lax.dot_general(dc, onehotT, (((0,), (0,)), ((), ())),
                              preferred_element_type=jnp.float32)
    ce_ref[0] = ceT

    idx_col = jnp.transpose(idxv)
    kio2 = jax.lax.broadcasted_iota(jnp.int32, (BATCH, DICT_SIZE), 1)
    oh_ref[slot] = (kio2 == idx_col).astype(jnp.float32)
    for cp in _copies(c, slot):
        cp.start()

    @pl.when(c == DIM_CODES - 1)
    def _drain():
        for j in range(N_SLOT):
            cc = DIM_CODES - N_SLOT + j
            for cp in _copies(cc, cc % N_SLOT):
                cp.wait()


def kernel(x, dictionary):
    xt = x.reshape(BATCH, DIM_CODES, DIM_EMBED).transpose(1, 2, 0)

    idx_t, ce_t, one_hot = pl.pallas_call(
        _fused_body,
        grid=(DIM_CODES,),
        in_specs=[
            pl.BlockSpec((1, DIM_EMBED, BATCH), lambda c: (c, 0, 0)),
            pl.BlockSpec((1, DICT_SIZE, DIM_EMBED), lambda c: (c, 0, 0)),
        ],
        out_specs=[
            pl.BlockSpec((1, 1, BATCH), lambda c: (c, 0, 0)),
            pl.BlockSpec((1, DIM_EMBED, BATCH), lambda c: (c, 0, 0)),
            pl.BlockSpec(memory_space=pltpu.MemorySpace.HBM),
        ],
        out_shape=[
            jax.ShapeDtypeStruct((DIM_CODES, 1, BATCH), jnp.int32),
            jax.ShapeDtypeStruct((DIM_CODES, DIM_EMBED, BATCH), jnp.float32),
            jax.ShapeDtypeStruct((BATCH, DIM_CODES, DICT_SIZE), jnp.float32),
        ],
        scratch_shapes=[
            pltpu.VMEM((N_SLOT, BATCH, DICT_SIZE), jnp.float32),
            pltpu.VMEM((DICT_SIZE, BATCH), jnp.int32),
            pltpu.SemaphoreType.DMA((N_SLOT, 4)),
        ],
    )(xt, dictionary)

    cw_e = ce_t.transpose(2, 0, 1).reshape(BATCH, DIM_CODES * DIM_EMBED)
    return cw_e, cw_e, one_hot
